# baseline (speedup 1.0000x reference)
import functools

import jax
import jax.numpy as jnp
from jax import lax
from jax.experimental import pallas as pl
from jax.experimental.pallas import tpu as pltpu
from jax.experimental.pallas import tpu_sc as plsc

N = 10000
E = 320000
D = 128

NC = 2
NS = 16
NW = NC * NS
EPW = E // NW
C = 80
ITERS = EPW // C
RPS = N // NS
ZFULL = RPS // C
ZREM = RPS - ZFULL * C

_mesh = plsc.VectorSubcoreMesh(
    core_axis_name="c", subcore_axis_name="s", num_cores=NC, num_subcores=NS
)


def _zero_fill(buf, rows, cols):
    zv = jnp.zeros((16,), jnp.float32)

    @pl.loop(0, rows)
    def _(r):
        @pl.loop(0, cols, step=16)
        def _(j):
            buf.at[r, pl.ds(j, 16)][...] = zv


ICH = EPW // C
NB = 3
IB = ICH
PHASES = ((0, ICH),)
IB_D = 32
PHASES_D = ((0, 32), (32, 32), (64, 32), (96, 29))


def _sc_aggr(x, row3, col3, do_deg):
    ib = IB_D if do_deg else IB
    phases = PHASES_D if do_deg else PHASES

    out_type = jax.ShapeDtypeStruct((NC, N, D), jnp.float32)
    scratch = [
        pltpu.VMEM((ib, C), jnp.int32),
        pltpu.VMEM((ib, C), jnp.int32),
    ] + [pltpu.VMEM((C, D), jnp.float32)] * NB + [
        pltpu.VMEM_SHARED((N, D), jnp.float32),
    ] + [pltpu.SemaphoreType.DMA] * (2 * NB)
    if do_deg:
        out_type = (out_type, jax.ShapeDtypeStruct((NC, N, 16), jnp.float32))
        scratch += [
            pltpu.VMEM((C, 16), jnp.float32),
            pltpu.VMEM_SHARED((N, 16), jnp.float32),
            pltpu.SemaphoreType.DMA,
        ]

    @functools.partial(
        pl.kernel,
        out_type=out_type,
        mesh=_mesh,
        scratch_types=scratch,
        compiler_params=pltpu.CompilerParams(use_tc_tiling_on_sc=False),
    )
    def k(x_hbm, row_hbm, col_hbm, *rest):
        if do_deg:
            (paggr_hbm, pdeg_hbm, idx_r, idx_c, *rest2) = rest
            ones_v, deg_sh, semd = rest2[3 * NB + 1:]
        else:
            (paggr_hbm, idx_r, idx_c, *rest2) = rest
        rows = rest2[:NB]
        aggr_sh = rest2[NB]
        semg = rest2[NB + 1:NB + 1 + NB]
        sems = rest2[NB + 1 + NB:NB + 1 + 2 * NB]

        c = lax.axis_index("c")
        s = lax.axis_index("s")
        wid = s * NC + c
        rs = s * RPS

        _zero_fill(rows[0], C, D)

        @pl.loop(0, ZFULL)
        def _(t):
            pltpu.sync_copy(rows[0], aggr_sh.at[pl.ds(rs + t * C, C)])

        pltpu.sync_copy(rows[0].at[pl.ds(0, ZREM)],
                        aggr_sh.at[pl.ds(rs + ZFULL * C, ZREM)])

        if do_deg:
            _zero_fill(ones_v, C, 16)

            @pl.loop(0, ZFULL)
            def _(t):
                pltpu.sync_copy(ones_v, deg_sh.at[pl.ds(rs + t * C, C)])

            pltpu.sync_copy(ones_v.at[pl.ds(0, ZREM)],
                            deg_sh.at[pl.ds(rs + ZFULL * C, ZREM)])

            ov = jnp.ones((16,), jnp.float32)

            @pl.loop(0, C)
            def _(r):
                ones_v.at[r][...] = ov

        plsc.subcore_barrier()

        def g_start(b, j):
            pltpu.async_copy(x_hbm.at[idx_r.at[j]], rows[b], semg[b])

        def g_wait(b, j):
            pltpu.make_async_copy(x_hbm.at[idx_r.at[j]], rows[b], semg[b]).wait()

        def s_start(b, j):
            pltpu.async_copy(rows[b], aggr_sh.at[idx_c.at[j]], sems[b], add=True)

        def s_wait(b, j):
            pltpu.make_async_copy(rows[b], aggr_sh.at[idx_c.at[j]], sems[b]).wait()

        def d_start(j):
            if do_deg:
                pltpu.async_copy(ones_v, deg_sh.at[idx_r.at[j]], semd, add=True)

        def d_wait(j):
            if do_deg:
                pltpu.make_async_copy(ones_v, deg_sh.at[idx_r.at[j]], semd).wait()

        for off, nchunks in phases:
            lo = min(off, ICH - ib)
            lb = off - lo
            pltpu.sync_copy(row_hbm.at[wid, pl.ds(lo, ib)], idx_r)
            pltpu.sync_copy(col_hbm.at[wid, pl.ds(lo, ib)], idx_c)

            FULL = nchunks // NB
            for b in range(NB):
                g_start(b, lb + b)

            @pl.loop(0, FULL - 1)
            def _(g):
                j = lb + NB * g
                for b in range(NB):
                    g_wait(b, j + b)
                    s_start(b, j + b)
                    d_start(j + b)
                for b in range(NB):
                    s_wait(b, j + b)
                    d_wait(j + b)
                    g_start(b, j + NB + b)

            jl = lb + NB * (FULL - 1)
            for b in range(NB):
                g_wait(b, jl + b)
                s_start(b, jl + b)
                d_start(jl + b)
            for b in range(NB):
                s_wait(b, jl + b)
                d_wait(jl + b)

            for j in range(lb + NB * FULL, lb + nchunks):
                pltpu.async_copy(x_hbm.at[idx_r.at[j]], rows[0], semg[0]).wait()
                pltpu.sync_copy(rows[0], aggr_sh.at[idx_c.at[j]], add=True)
                d_start(j)
                d_wait(j)

        plsc.subcore_barrier()
        pltpu.sync_copy(aggr_sh.at[pl.ds(rs, RPS)],
                        paggr_hbm.at[c, pl.ds(rs, RPS)])
        if do_deg:
            pltpu.sync_copy(deg_sh.at[pl.ds(rs, RPS)],
                            pdeg_hbm.at[c, pl.ds(rs, RPS)])

    return k(x, row3, col3)


BR = 2000

_tc_params = pltpu.CompilerParams(dimension_semantics=("parallel",))


def _combine_body(relu, p_ref, d_ref, x_ref, w_ref, b_ref, o_ref):
    d = d_ref[0, :, 0:1] + d_ref[1, :, 0:1]
    inv = 1.0 / jnp.maximum(d, 1.0)
    a = (p_ref[0] + p_ref[1]) * inv + x_ref[...]
    y = jnp.dot(a, w_ref[...], preferred_element_type=jnp.float32) + b_ref[...]
    o_ref[...] = jnp.maximum(y, 0.0) if relu else y


def _combine(p, degp, x, w, b, relu):
    return pl.pallas_call(
        functools.partial(_combine_body, relu),
        grid=(N // BR,),
        compiler_params=_tc_params,
        in_specs=[
            pl.BlockSpec((NC, BR, D), lambda i: (0, i, 0)),
            pl.BlockSpec((NC, BR, 16), lambda i: (0, i, 0)),
            pl.BlockSpec((BR, D), lambda i: (i, 0)),
            pl.BlockSpec((D, D), lambda i: (0, 0)),
            pl.BlockSpec((1, D), lambda i: (0, 0)),
        ],
        out_specs=pl.BlockSpec((BR, D), lambda i: (i, 0)),
        out_shape=jax.ShapeDtypeStruct((N, D), jnp.float32),
    )(p, degp, x, w, b.reshape(1, D))


def kernel(x, edge_index, W0, b0, W1, b1, W2, b2):
    ei = edge_index.astype(jnp.int32)
    row = ei[0]
    col = ei[1]
    row3 = row.reshape(NW, ICH, C)
    col3 = col.reshape(NW, ICH, C)

    p1, degp = _sc_aggr(x, row3, col3, do_deg=True)
    h1 = _combine(p1, degp, x, W0, b0, relu=True)
    p2 = _sc_aggr(h1, row3, col3, do_deg=False)
    h2 = _combine(p2, degp, h1, W1, b1, relu=True)
    p3 = _sc_aggr(h2, row3, col3, do_deg=False)
    return _combine(p3, degp, h2, W2, b2, relu=False)
